# baseline (speedup 1.0000x reference)
import functools

import jax
import jax.numpy as jnp
import numpy as np
from jax.experimental import pallas as pl
from jax.experimental.pallas import tpu as pltpu
from jax.sharding import Mesh, PartitionSpec as P

try:
    from jax.experimental.shard_map import shard_map
except ImportError:
    from jax import shard_map

B, D, T, K = 16, 256, 1024, 1024
R = 128
COMMITMENT_COST = 0.25


def _vq_body(z_ref, w_ref, wb_ref, wsq_ref, codes_ref, zq_ref, loss_ref):
    z = z_ref[0]
    w = w_ref[...]
    wb = wb_ref[...]
    wsq = wsq_ref[...]
    zsq = jnp.sum(z * z, axis=0, keepdims=True)
    m = jax.lax.dot_general(w, z, (((1,), (0,)), ((), ())),
                            preferred_element_type=jnp.float32)

    iota_r = jax.lax.broadcasted_iota(jnp.int32, (R, T), 0).astype(jnp.float32)
    val = (zsq - 2.0 * m[0:R, :]) + wsq[0:R, :]
    idx = iota_r
    for i in range(1, K // R):
        d = (zsq - 2.0 * m[i * R:(i + 1) * R, :]) + wsq[i * R:(i + 1) * R, :]
        take = d < val
        val = jnp.where(take, d, val)
        idx = jnp.where(take, iota_r + jnp.float32(i * R), idx)
    s = R // 2
    while s >= 1:
        av, bv = val[:s], val[s:2 * s]
        ai, bi = idx[:s], idx[s:2 * s]
        take = (bv < av) | ((bv == av) & (bi < ai))
        val = jnp.where(take, bv, av)
        idx = jnp.where(take, bi, ai)
        s //= 2
    codes = idx.astype(jnp.int32)
    codes_ref[0] = codes
    onehot = (jax.lax.broadcasted_iota(jnp.int32, (K, T), 0)
              == codes).astype(jnp.bfloat16)
    zq = jax.lax.dot_general(wb, onehot, (((0,), (0,)), ((), ())),
                             preferred_element_type=jnp.float32)
    zq_ref[0] = zq
    loss_ref[0, 0, :] = jnp.full((128,), jnp.sum(val), jnp.float32)


def _vq_block(z, W, Wb, wsq):
    nb = z.shape[0]
    return pl.pallas_call(
        _vq_body,
        grid=(nb,),
        in_specs=[
            pl.BlockSpec((1, D, T), lambda b: (b, 0, 0)),
            pl.BlockSpec((K, D), lambda b: (0, 0)),
            pl.BlockSpec((K, D), lambda b: (0, 0)),
            pl.BlockSpec((K, 1), lambda b: (0, 0)),
        ],
        out_specs=[
            pl.BlockSpec((1, 1, T), lambda b: (b, 0, 0)),
            pl.BlockSpec((1, D, T), lambda b: (b, 0, 0)),
            pl.BlockSpec((1, 1, 128), lambda b: (b, 0, 0)),
        ],
        out_shape=[
            jax.ShapeDtypeStruct((nb, 1, T), jnp.int32),
            jax.ShapeDtypeStruct((nb, D, T), jnp.float32),
            jax.ShapeDtypeStruct((nb, 1, 128), jnp.float32),
        ],
        compiler_params=pltpu.CompilerParams(
            dimension_semantics=("arbitrary",),
        ),
    )(z, W, Wb, wsq)


_DEVS = jax.devices()
_NDEV = 2 if len(_DEVS) >= 2 else 1
_MESH = Mesh(np.array(_DEVS[:_NDEV]), ("x",))


@functools.partial(jax.jit, static_argnames=())
def kernel(z, W):
    Wb = W.astype(jnp.bfloat16)
    wsq = jnp.sum(W ** 2, axis=1).reshape(K, 1)
    sharded = shard_map(
        _vq_block, mesh=_MESH,
        in_specs=(P("x"), P(), P(), P()),
        out_specs=(P("x"), P("x"), P("x")),
        check_rep=False,
    )
    codes3, zq, loss_parts = sharded(z, W, Wb, wsq)
    codes = codes3.reshape(B * T)
    sq_err_sum = jnp.sum(loss_parts[:, 0, 0])
    vq_loss = (1.0 + COMMITMENT_COST) * sq_err_sum / (B * D * T)
    return zq, vq_loss, codes


# speedup vs baseline: 5.6658x; 5.6658x over previous
import functools

import jax
import jax.numpy as jnp
import numpy as np
from jax.experimental import pallas as pl
from jax.experimental.pallas import tpu as pltpu
from jax.sharding import Mesh, PartitionSpec as P

try:
    from jax.experimental.shard_map import shard_map
except ImportError:
    from jax import shard_map

B, D, T, K = 16, 256, 1024, 1024
R = 128
COMMITMENT_COST = 0.25


def _vq_body(z_ref, w_ref, wb_ref, wsq_ref, zsq_ref, codes_ref, zq_ref,
             loss_ref):
    z = z_ref[0]
    w = w_ref[...]
    wb = wb_ref[...]
    wsq = wsq_ref[...]
    zsq = zsq_ref[0]
    m = jax.lax.dot_general(w, z, (((1,), (0,)), ((), ())),
                            preferred_element_type=jnp.float32)

    iota_r = jax.lax.broadcasted_iota(jnp.int32, (R, T), 0).astype(jnp.float32)
    val = (zsq - 2.0 * m[0:R, :]) + wsq[0:R, :]
    idx = iota_r
    for i in range(1, K // R):
        d = (zsq - 2.0 * m[i * R:(i + 1) * R, :]) + wsq[i * R:(i + 1) * R, :]
        take = d < val
        val = jnp.where(take, d, val)
        idx = jnp.where(take, iota_r + jnp.float32(i * R), idx)
    s = R // 2
    while s >= 1:
        av, bv = val[:s], val[s:2 * s]
        ai, bi = idx[:s], idx[s:2 * s]
        take = (bv < av) | ((bv == av) & (bi < ai))
        val = jnp.where(take, bv, av)
        idx = jnp.where(take, bi, ai)
        s //= 2
    codes = idx.astype(jnp.int32)
    codes_ref[0] = codes
    onehot = (jax.lax.broadcasted_iota(jnp.int32, (K, T), 0)
              == codes).astype(jnp.bfloat16)
    zq = jax.lax.dot_general(wb, onehot, (((0,), (0,)), ((), ())),
                             preferred_element_type=jnp.float32)
    zq_ref[0] = zq
    loss_ref[0, 0, :] = jnp.full((128,), jnp.sum(val), jnp.float32)


def _vq_block(z, W, Wb, wsq, zsq):
    nb = z.shape[0]
    return pl.pallas_call(
        _vq_body,
        grid=(nb,),
        in_specs=[
            pl.BlockSpec((1, D, T), lambda b: (b, 0, 0)),
            pl.BlockSpec((K, D), lambda b: (0, 0)),
            pl.BlockSpec((K, D), lambda b: (0, 0)),
            pl.BlockSpec((K, 1), lambda b: (0, 0)),
            pl.BlockSpec((1, 1, T), lambda b: (b, 0, 0)),
        ],
        out_specs=[
            pl.BlockSpec((1, 1, T), lambda b: (b, 0, 0)),
            pl.BlockSpec((1, D, T), lambda b: (b, 0, 0)),
            pl.BlockSpec((1, 1, 128), lambda b: (b, 0, 0)),
        ],
        out_shape=[
            jax.ShapeDtypeStruct((nb, 1, T), jnp.int32),
            jax.ShapeDtypeStruct((nb, D, T), jnp.float32),
            jax.ShapeDtypeStruct((nb, 1, 128), jnp.float32),
        ],
        compiler_params=pltpu.CompilerParams(
            dimension_semantics=("arbitrary",),
        ),
    )(z, W, Wb, wsq, zsq)


@functools.partial(jax.jit, static_argnames=())
def kernel(z, W):
    Wb = W.astype(jnp.bfloat16)
    wsq = jax.lax.optimization_barrier(jnp.sum(W ** 2, axis=1)).reshape(K, 1)
    z_flat = jax.lax.optimization_barrier(
        jnp.transpose(z, (0, 2, 1)).reshape(-1, D))
    zsq = jax.lax.optimization_barrier(
        jnp.sum(z_flat ** 2, axis=1)).reshape(B, 1, T)
    codes3, zq, loss_parts = _vq_block(z, W, Wb, wsq, zsq)
    codes = codes3.reshape(B * T)
    sq_err_sum = jnp.sum(loss_parts[:, 0, 0])
    vq_loss = (1.0 + COMMITMENT_COST) * sq_err_sum / (B * D * T)
    return zq, vq_loss, codes


# speedup vs baseline: 5.6676x; 1.0003x over previous
import functools

import jax
import jax.numpy as jnp
from jax.experimental import pallas as pl
from jax.experimental.pallas import tpu as pltpu

B, D, T, K = 16, 256, 1024, 1024
R = 128
COMMITMENT_COST = 0.25


def _vq_body(z_ref, w_ref, wb_ref, wsq_ref, zsq_ref, codes_ref, zq_ref,
             loss_ref):
    z = z_ref[0]
    w = w_ref[...]
    wb = wb_ref[...]
    wsq = wsq_ref[...]
    zsq = zsq_ref[0]
    m = jax.lax.dot_general(w, z, (((1,), (0,)), ((), ())),
                            preferred_element_type=jnp.float32)

    iota_r = jax.lax.broadcasted_iota(jnp.int32, (R, T), 0).astype(jnp.float32)
    val = (zsq - 2.0 * m[0:R, :]) + wsq[0:R, :]
    idx = iota_r
    for i in range(1, K // R):
        d = (zsq - 2.0 * m[i * R:(i + 1) * R, :]) + wsq[i * R:(i + 1) * R, :]
        take = d < val
        val = jnp.where(take, d, val)
        idx = jnp.where(take, iota_r + jnp.float32(i * R), idx)
    s = R // 2
    while s >= 1:
        av, bv = val[:s], val[s:2 * s]
        ai, bi = idx[:s], idx[s:2 * s]
        take = (bv < av) | ((bv == av) & (bi < ai))
        val = jnp.where(take, bv, av)
        idx = jnp.where(take, bi, ai)
        s //= 2
    codes = idx.astype(jnp.int32)
    codes_ref[0] = codes
    onehot = (jax.lax.broadcasted_iota(jnp.int32, (K, T), 0)
              == codes).astype(jnp.bfloat16)
    zq = jax.lax.dot_general(wb, onehot, (((0,), (0,)), ((), ())),
                             preferred_element_type=jnp.float32)
    zq_ref[0] = zq
    loss_ref[0, 0, :] = jnp.full((128,), jnp.sum(val), jnp.float32)


def _vq_block(z, W, Wb, wsq, zsq):
    nb = z.shape[0]
    return pl.pallas_call(
        _vq_body,
        grid=(nb,),
        in_specs=[
            pl.BlockSpec((1, D, T), lambda b: (b, 0, 0)),
            pl.BlockSpec((K, D), lambda b: (0, 0)),
            pl.BlockSpec((K, D), lambda b: (0, 0)),
            pl.BlockSpec((K, 1), lambda b: (0, 0)),
            pl.BlockSpec((1, 1, T), lambda b: (b, 0, 0)),
        ],
        out_specs=[
            pl.BlockSpec((1, 1, T), lambda b: (b, 0, 0)),
            pl.BlockSpec((1, D, T), lambda b: (b, 0, 0)),
            pl.BlockSpec((1, 1, 128), lambda b: (b, 0, 0)),
        ],
        out_shape=[
            jax.ShapeDtypeStruct((nb, 1, T), jnp.int32),
            jax.ShapeDtypeStruct((nb, D, T), jnp.float32),
            jax.ShapeDtypeStruct((nb, 1, 128), jnp.float32),
        ],
        compiler_params=pltpu.CompilerParams(
            dimension_semantics=("arbitrary",),
        ),
    )(z, W, Wb, wsq, zsq)


@functools.partial(jax.jit, static_argnames=())
def kernel(z, W):
    Wb = W.astype(jnp.bfloat16)
    wsq = jax.lax.optimization_barrier(jnp.sum(W ** 2, axis=1)).reshape(K, 1)
    z_flat = jax.lax.optimization_barrier(
        jnp.transpose(z, (0, 2, 1)).reshape(-1, D))
    zsq = jax.lax.optimization_barrier(
        jnp.sum(z_flat ** 2, axis=1)).reshape(B, 1, T)
    codes3, zq, loss_parts = _vq_block(z, W, Wb, wsq, zsq)
    codes = codes3.reshape(B * T)
    sq_err_sum = jnp.sum(loss_parts[:, 0, 0])
    vq_loss = (1.0 + COMMITMENT_COST) * sq_err_sum / (B * D * T)
    return zq, vq_loss, codes
